# Initial kernel scaffold; baseline (speedup 1.0000x reference)
import jax
import jax.numpy as jnp
from jax.experimental import pallas as pl


def kernel(node_type, node_tokens, children_index, subtree_labels, type_table, token_table, Wt, Wl, Wr, conv_b, softmax_w, softmax_b):
    raise NotImplementedError("write your pallas kernel here")


# speedup vs baseline: 12.4100x; 12.4100x over previous
import functools

import jax
import jax.numpy as jnp
from jax import lax
from jax.experimental import pallas as pl
from jax.experimental.pallas import tpu as pltpu
from jax.experimental.pallas import tpu_sc as plsc

B, N, C, T = 64, 256, 8, 8
DH = 128
D = 256
NUM_SUB = 5000
M = B * N

NC, NS = 2, 16
NW = NC * NS
NODES_PER_W = M // NW
NODES_PER_STEP = 128 // T
STEPS = NODES_PER_W // NODES_PER_STEP


def _sc_token_mean(token_table, flat_tok):
    mesh = plsc.VectorSubcoreMesh(core_axis_name="c", subcore_axis_name="s")

    @functools.partial(
        pl.kernel,
        out_type=jax.ShapeDtypeStruct((M, DH), jnp.float32),
        mesh=mesh,
        scratch_types=[
            pltpu.VMEM((128,), jnp.int32),
            pltpu.VMEM((128, DH), jnp.float32),
            pltpu.VMEM((NODES_PER_STEP, DH), jnp.float32),
            pltpu.SemaphoreType.DMA,
        ],
    )
    def k(table_hbm, tok_hbm, out_hbm, idx_v, rows_v, out_v, sem):
        wid = lax.axis_index("s") * NC + lax.axis_index("c")
        base_node = wid * NODES_PER_W

        def step(s, carry):
            node0 = base_node + s * NODES_PER_STEP
            pltpu.sync_copy(tok_hbm.at[pl.ds(node0 * T, 128)], idx_v)
            pltpu.async_copy(table_hbm.at[idx_v], rows_v, sem).wait()
            for g in range(NODES_PER_STEP):
                for j in range(DH // 16):
                    sl = pl.ds(j * 16, 16)
                    acc = rows_v[g * T, sl]
                    for t in range(1, T):
                        acc = acc + rows_v[g * T + t, sl]
                    out_v[g, sl] = acc * (1.0 / T)
            pltpu.sync_copy(out_v, out_hbm.at[pl.ds(node0, NODES_PER_STEP)])
            return carry

        lax.fori_loop(0, STEPS, step, 0)

    return k(token_table, flat_tok)


BB = 8
GRID = B // BB


def _tc_body(tok_ref, ntype_ref, cidx_ref, lab_ref, ttab_ref,
             wt_ref, wl_ref, wr_ref, cb_ref, swt_ref, sb_ref,
             logits_ref, loss_ref):
    i = pl.program_id(0)

    ntype = ntype_ref[...][..., None]
    oh = (ntype == lax.broadcasted_iota(jnp.int32, (1, 1, DH), 2)
          ).astype(jnp.float32).reshape(BB * N, DH)
    type_emb = jnp.dot(oh, ttab_ref[...], preferred_element_type=jnp.float32)
    node_emb = jnp.concatenate(
        [type_emb, tok_ref[...].reshape(BB * N, DH)], axis=-1)

    cidx = cidx_ref[...]
    mask = (cidx > 0).astype(jnp.float32)
    n_c = jnp.sum(mask, axis=-1, keepdims=True)
    pos = lax.broadcasted_iota(jnp.int32, (1, 1, C), 2).astype(jnp.float32) + 1.0
    denom = jnp.maximum(n_c - 1.0, 1.0)
    ratio = (pos - 1.0) / denom
    ratio = jnp.where(n_c == 1.0, 0.5, ratio)
    eta_r = mask * ratio
    eta_l = mask * (1.0 - ratio)

    ne3 = node_emb.reshape(BB, N, D)
    iota_n = lax.broadcasted_iota(jnp.int32, (N, N), 1)
    lefts, rights = [], []
    for bb in range(BB):
        a_l = jnp.zeros((N, N), jnp.float32)
        a_r = jnp.zeros((N, N), jnp.float32)
        for c in range(C):
            ohc = (cidx[bb][:, c:c + 1] == iota_n).astype(jnp.float32)
            a_l = a_l + eta_l[bb][:, c:c + 1] * ohc
            a_r = a_r + eta_r[bb][:, c:c + 1] * ohc
        lefts.append(jnp.dot(a_l, ne3[bb], preferred_element_type=jnp.float32))
        rights.append(jnp.dot(a_r, ne3[bb], preferred_element_type=jnp.float32))
    left_all = jnp.concatenate(lefts, axis=0)
    right_all = jnp.concatenate(rights, axis=0)

    h = (jnp.dot(node_emb, wt_ref[...], preferred_element_type=jnp.float32)
         + jnp.dot(left_all, wl_ref[...], preferred_element_type=jnp.float32)
         + jnp.dot(right_all, wr_ref[...], preferred_element_type=jnp.float32)
         + cb_ref[...])
    conv = jnp.tanh(h)
    cv = jnp.max(conv.reshape(BB, N, D), axis=1)

    logits = (jnp.dot(cv, swt_ref[...], preferred_element_type=jnp.float32)
              + sb_ref[...])
    logits_ref[...] = logits

    lab = lab_ref[...]
    bce = (jnp.maximum(logits, 0.0) - logits * lab
           + jnp.log1p(jnp.exp(-jnp.abs(logits))))
    partial = jnp.sum(jnp.sum(bce, axis=1, keepdims=True), axis=0, keepdims=True)

    @pl.when(i == 0)
    def _():
        loss_ref[...] = jnp.zeros((1, 1), jnp.float32)

    loss_ref[...] += partial

    @pl.when(i == GRID - 1)
    def _():
        loss_ref[...] = loss_ref[...] * (1.0 / (B * NUM_SUB))


def _tc_call(tok_emb, node_type, children_index, subtree_labels,
             ttab_pad, Wt, Wl, Wr, conv_b, sw_t, softmax_b):
    out_shape = [
        jax.ShapeDtypeStruct((B, NUM_SUB), jnp.float32),
        jax.ShapeDtypeStruct((1, 1), jnp.float32),
    ]
    logits, loss = pl.pallas_call(
        _tc_body,
        grid=(GRID,),
        in_specs=[
            pl.BlockSpec((BB, N, DH), lambda i: (i, 0, 0)),
            pl.BlockSpec((BB, N), lambda i: (i, 0)),
            pl.BlockSpec((BB, N, C), lambda i: (i, 0, 0)),
            pl.BlockSpec((BB, NUM_SUB), lambda i: (i, 0)),
            pl.BlockSpec((DH, DH), lambda i: (0, 0)),
            pl.BlockSpec((D, D), lambda i: (0, 0)),
            pl.BlockSpec((D, D), lambda i: (0, 0)),
            pl.BlockSpec((D, D), lambda i: (0, 0)),
            pl.BlockSpec((1, D), lambda i: (0, 0)),
            pl.BlockSpec((D, NUM_SUB), lambda i: (0, 0)),
            pl.BlockSpec((1, NUM_SUB), lambda i: (0, 0)),
        ],
        out_specs=[
            pl.BlockSpec((BB, NUM_SUB), lambda i: (i, 0)),
            pl.BlockSpec((1, 1), lambda i: (0, 0)),
        ],
        out_shape=out_shape,
    )(tok_emb, node_type, children_index, subtree_labels,
      ttab_pad, Wt, Wl, Wr, conv_b, sw_t, softmax_b)
    return logits, loss


def kernel(node_type, node_tokens, children_index, subtree_labels,
           type_table, token_table, Wt, Wl, Wr, conv_b, softmax_w, softmax_b):
    node_type = node_type.astype(jnp.int32)
    children_index = children_index.astype(jnp.int32)
    flat_tok = node_tokens.astype(jnp.int32).reshape(M * T)

    tok_emb = _sc_token_mean(token_table, flat_tok)
    tok_emb = tok_emb.reshape(B, N, DH)

    ttab_pad = jnp.zeros((DH, DH), jnp.float32).at[:type_table.shape[0]].set(type_table)
    sw_t = softmax_w.T

    logits, loss = _tc_call(
        tok_emb, node_type, children_index, subtree_labels,
        ttab_pad, Wt, Wl, Wr, conv_b.reshape(1, D), sw_t,
        softmax_b.reshape(1, NUM_SUB))
    return (loss[0, 0], logits)


# speedup vs baseline: 19.1437x; 1.5426x over previous
import functools

import jax
import jax.numpy as jnp
import numpy as np
from jax import lax
from jax.experimental import pallas as pl
from jax.experimental.pallas import tpu as pltpu
from jax.experimental.pallas import tpu_sc as plsc

B, N, C, T = 64, 256, 8, 8
DH = 128
D = 256
NUM_SUB = 5000
M = B * N

NC, NS = 2, 16
NW = NC * NS
NODES_PER_W = M // NW
NODES_PER_STEP = 128 // T
STEPS = NODES_PER_W // NODES_PER_STEP


def _sc_token_mean(token_table, tok2d):
    mesh = plsc.VectorSubcoreMesh(core_axis_name="c", subcore_axis_name="s")

    @functools.partial(
        pl.kernel,
        out_type=jax.ShapeDtypeStruct((M, DH), jnp.float32),
        mesh=mesh,
        scratch_types=[
            pltpu.VMEM((STEPS, 128), jnp.int32),
            pltpu.VMEM((2, 128, DH), jnp.float32),
            pltpu.VMEM((2, NODES_PER_STEP, DH), jnp.float32),
            pltpu.SemaphoreType.DMA,
            pltpu.SemaphoreType.DMA,
        ],
    )
    def k(table_hbm, tok_hbm, out_hbm, idx_all, rows_v, out_v, sem0, sem1):
        wid = lax.axis_index("s") * NC + lax.axis_index("c")
        base_node = wid * NODES_PER_W
        sems = (sem0, sem1)

        pltpu.sync_copy(tok_hbm.at[pl.ds(wid * STEPS, STEPS)], idx_all)

        def start_gather(s, buf):
            pltpu.async_copy(table_hbm.at[idx_all.at[s]],
                             rows_v.at[buf], sems[buf])

        def wait_gather(s, buf):
            pltpu.make_async_copy(table_hbm.at[idx_all.at[s]],
                                  rows_v.at[buf], sems[buf]).wait()

        def reduce_and_store(s, buf):
            for g in range(NODES_PER_STEP):
                for j in range(DH // 16):
                    sl = pl.ds(j * 16, 16)
                    r = [rows_v[buf, g * T + t, sl] for t in range(T)]
                    acc = (((r[0] + r[1]) + (r[2] + r[3]))
                           + ((r[4] + r[5]) + (r[6] + r[7])))
                    out_v[buf, g, sl] = acc * (1.0 / T)
            pltpu.sync_copy(
                out_v.at[buf],
                out_hbm.at[pl.ds(base_node + s * NODES_PER_STEP,
                                 NODES_PER_STEP)])

        start_gather(0, 0)

        def pair(p, carry):
            s0 = p * 2
            start_gather(s0 + 1, 1)
            wait_gather(s0, 0)
            reduce_and_store(s0, 0)

            @pl.when(s0 + 2 < STEPS)
            def _():
                start_gather(s0 + 2, 0)

            wait_gather(s0 + 1, 1)
            reduce_and_store(s0 + 1, 1)
            return carry

        lax.fori_loop(0, STEPS // 2, pair, 0)

    return k(token_table, tok2d)


BB = 8
GRID = B // BB


def _tc_pre_body(ntype_ref, cidx_ref, ttab_ref, te_ref, alr_ref):
    ntype = ntype_ref[...][..., None]
    oh = (ntype == lax.broadcasted_iota(jnp.int32, (1, 1, DH), 2)
          ).astype(jnp.bfloat16).reshape(BB * N, DH)
    te = jnp.dot(oh, ttab_ref[...], preferred_element_type=jnp.float32)
    te_ref[...] = te.astype(jnp.bfloat16).reshape(BB, N, DH)

    cidx = cidx_ref[...]
    mask = (cidx > 0).astype(jnp.float32)
    n_c = jnp.sum(mask, axis=-1, keepdims=True)
    pos = lax.broadcasted_iota(jnp.int32, (1, 1, C), 2).astype(jnp.float32) + 1.0
    denom = jnp.maximum(n_c - 1.0, 1.0)
    ratio = (pos - 1.0) / denom
    ratio = jnp.where(n_c == 1.0, 0.5, ratio)
    eta_r = (mask * ratio).astype(jnp.bfloat16)
    eta_l = (mask * (1.0 - ratio)).astype(jnp.bfloat16)

    cidx_bf = cidx.astype(jnp.bfloat16)
    iota_n = lax.broadcasted_iota(jnp.int32, (N, N), 1).astype(jnp.bfloat16)
    zero = jnp.zeros((N, N), jnp.bfloat16)
    for bb in range(BB):
        a_l = zero
        a_r = zero
        for c in range(C):
            hit = cidx_bf[bb][:, c:c + 1] == iota_n
            a_l = a_l + jnp.where(hit, eta_l[bb][:, c:c + 1], zero)
            a_r = a_r + jnp.where(hit, eta_r[bb][:, c:c + 1], zero)
        alr_ref[bb, 0:N, :] = a_l
        alr_ref[bb, N:2 * N, :] = a_r


def _tc_main_body(tok_ref, te_ref, alr_ref, lab_ref, wcat_ref, cb_ref,
                  sw_ref, sb_ref, logits_ref, loss_ref):
    i = pl.program_id(0)
    ne3 = jnp.concatenate([te_ref[...], tok_ref[...].astype(jnp.bfloat16)],
                          axis=-1)

    parts = []
    for bb in range(BB):
        agg = jnp.dot(alr_ref[bb], ne3[bb], preferred_element_type=jnp.float32)
        parts.append(jnp.concatenate(
            [ne3[bb], agg[:N].astype(jnp.bfloat16),
             agg[N:].astype(jnp.bfloat16)], axis=1))
    x_all = jnp.concatenate(parts, axis=0)

    h = (jnp.dot(x_all, wcat_ref[...], preferred_element_type=jnp.float32)
         + cb_ref[...])
    conv = jnp.tanh(h)
    cv = jnp.max(conv.reshape(BB, N, D), axis=1)

    logits = (lax.dot_general(cv, sw_ref[...], (((1,), (1,)), ((), ())),
                              preferred_element_type=jnp.float32)
              + sb_ref[...])
    logits_ref[...] = logits

    lab = lab_ref[...]
    bce = (jnp.maximum(logits, 0.0) - logits * lab
           + jnp.log1p(jnp.exp(-jnp.abs(logits))))
    partial = jnp.sum(jnp.sum(bce, axis=1, keepdims=True), axis=0, keepdims=True)

    @pl.when(i == 0)
    def _():
        loss_ref[...] = jnp.zeros((1, 1), jnp.float32)

    loss_ref[...] += partial

    @pl.when(i == GRID - 1)
    def _():
        loss_ref[...] = loss_ref[...] * (1.0 / (B * NUM_SUB))


def _tc_pre_call(node_type, children_index, ttab_pad):
    return pl.pallas_call(
        _tc_pre_body,
        grid=(GRID,),
        in_specs=[
            pl.BlockSpec((BB, N), lambda i: (i, 0)),
            pl.BlockSpec((BB, N, C), lambda i: (i, 0, 0)),
            pl.BlockSpec((DH, DH), lambda i: (0, 0)),
        ],
        out_specs=[
            pl.BlockSpec((BB, N, DH), lambda i: (i, 0, 0)),
            pl.BlockSpec((BB, 2 * N, N), lambda i: (i, 0, 0)),
        ],
        out_shape=[
            jax.ShapeDtypeStruct((B, N, DH), jnp.bfloat16),
            jax.ShapeDtypeStruct((B, 2 * N, N), jnp.bfloat16),
        ],
    )(node_type, children_index, ttab_pad)


def _tc_main_call(tok_emb, te, alr, subtree_labels, w_cat, conv_b,
                  softmax_w, softmax_b):
    return pl.pallas_call(
        _tc_main_body,
        grid=(GRID,),
        in_specs=[
            pl.BlockSpec((BB, N, DH), lambda i: (i, 0, 0)),
            pl.BlockSpec((BB, N, DH), lambda i: (i, 0, 0)),
            pl.BlockSpec((BB, 2 * N, N), lambda i: (i, 0, 0)),
            pl.BlockSpec((BB, NUM_SUB), lambda i: (i, 0)),
            pl.BlockSpec((3 * D, D), lambda i: (0, 0)),
            pl.BlockSpec((1, D), lambda i: (0, 0)),
            pl.BlockSpec((NUM_SUB, D), lambda i: (0, 0)),
            pl.BlockSpec((1, NUM_SUB), lambda i: (0, 0)),
        ],
        out_specs=[
            pl.BlockSpec((BB, NUM_SUB), lambda i: (i, 0)),
            pl.BlockSpec((1, 1), lambda i: (0, 0)),
        ],
        out_shape=[
            jax.ShapeDtypeStruct((B, NUM_SUB), jnp.float32),
            jax.ShapeDtypeStruct((1, 1), jnp.float32),
        ],
    )(tok_emb, te, alr, subtree_labels, w_cat, conv_b, softmax_w, softmax_b)


def kernel(node_type, node_tokens, children_index, subtree_labels,
           type_table, token_table, Wt, Wl, Wr, conv_b, softmax_w, softmax_b):
    node_type = node_type.astype(jnp.int32)
    children_index = children_index.astype(jnp.int32)
    tok2d = node_tokens.astype(jnp.int32).reshape(M * T // 128, 128)

    tok_emb = _sc_token_mean(token_table, tok2d)
    tok_emb = tok_emb.reshape(B, N, DH)

    ttab_pad = jnp.zeros((DH, DH), jnp.float32).at[:type_table.shape[0]].set(
        type_table).astype(jnp.bfloat16)
    w_cat = jnp.concatenate([Wt, Wl, Wr], axis=0).astype(jnp.bfloat16)

    te, alr = _tc_pre_call(node_type, children_index, ttab_pad)
    logits, loss = _tc_main_call(
        tok_emb, te, alr, subtree_labels, w_cat, conv_b.reshape(1, D),
        softmax_w, softmax_b.reshape(1, NUM_SUB))
    return (loss[0, 0], logits)


# speedup vs baseline: 20.0099x; 1.0452x over previous
import functools

import jax
import jax.numpy as jnp
import numpy as np
from jax import lax
from jax.experimental import pallas as pl
from jax.experimental.pallas import tpu as pltpu
from jax.experimental.pallas import tpu_sc as plsc

B, N, C, T = 64, 256, 8, 8
DH = 128
D = 256
NUM_SUB = 5000
M = B * N

NC, NS = 2, 16
NW = NC * NS
NODES_PER_W = M // NW
NODES_PER_STEP = 128 // T
STEPS = NODES_PER_W // NODES_PER_STEP


def _sc_token_mean(token_table, tok2d):
    mesh = plsc.VectorSubcoreMesh(core_axis_name="c", subcore_axis_name="s")

    @functools.partial(
        pl.kernel,
        out_type=jax.ShapeDtypeStruct((M, DH), jnp.float32),
        mesh=mesh,
        scratch_types=[
            pltpu.VMEM((STEPS, 128), jnp.int32),
            pltpu.VMEM((2, 128, DH), jnp.float32),
            pltpu.VMEM((2, NODES_PER_STEP, DH), jnp.float32),
            pltpu.SemaphoreType.DMA,
            pltpu.SemaphoreType.DMA,
            pltpu.SemaphoreType.DMA,
            pltpu.SemaphoreType.DMA,
        ],
    )
    def k(table_hbm, tok_hbm, out_hbm, idx_all, rows_v, out_v,
          sem0, sem1, osem0, osem1):
        wid = lax.axis_index("s") * NC + lax.axis_index("c")
        base_node = wid * NODES_PER_W
        sems = (sem0, sem1)
        osems = (osem0, osem1)

        pltpu.sync_copy(tok_hbm.at[pl.ds(wid * STEPS, STEPS)], idx_all)

        def start_gather(s, buf):
            pltpu.async_copy(table_hbm.at[idx_all.at[s]],
                             rows_v.at[buf], sems[buf])

        def wait_gather(s, buf):
            pltpu.make_async_copy(table_hbm.at[idx_all.at[s]],
                                  rows_v.at[buf], sems[buf]).wait()

        def reduce_and_store(s, buf):
            for g in range(NODES_PER_STEP):
                for j in range(DH // 16):
                    sl = pl.ds(j * 16, 16)
                    acc = rows_v[buf, g * T, sl]
                    for t in range(1, T):
                        acc = acc + rows_v[buf, g * T + t, sl]
                    out_v[buf, g, sl] = acc * (1.0 / T)
            pltpu.async_copy(
                out_v.at[buf],
                out_hbm.at[pl.ds(base_node + s * NODES_PER_STEP,
                                 NODES_PER_STEP)], osems[buf])

        def wait_out(s, buf):
            pltpu.make_async_copy(
                out_v.at[buf],
                out_hbm.at[pl.ds(base_node + s * NODES_PER_STEP,
                                 NODES_PER_STEP)], osems[buf]).wait()

        start_gather(0, 0)

        def pair(p, carry):
            s0 = p * 2
            start_gather(s0 + 1, 1)
            wait_gather(s0, 0)

            @pl.when(p >= 1)
            def _():
                wait_out(s0 - 2, 0)

            reduce_and_store(s0, 0)

            @pl.when(s0 + 2 < STEPS)
            def _():
                start_gather(s0 + 2, 0)

            wait_gather(s0 + 1, 1)

            @pl.when(p >= 1)
            def _():
                wait_out(s0 - 1, 1)

            reduce_and_store(s0 + 1, 1)
            return carry

        lax.fori_loop(0, STEPS // 2, pair, 0)
        wait_out(STEPS - 2, 0)
        wait_out(STEPS - 1, 1)

    return k(token_table, tok2d)


BB = 8
GRID = B // BB


def _tc_pre_body(ntype_ref, cidx_ref, ttab_ref, te_ref, alr_ref):
    ntype = ntype_ref[...][..., None]
    oh = (ntype == lax.broadcasted_iota(jnp.int32, (1, 1, DH), 2)
          ).astype(jnp.bfloat16).reshape(BB * N, DH)
    te = jnp.dot(oh, ttab_ref[...], preferred_element_type=jnp.float32)
    te_ref[...] = te.astype(jnp.bfloat16).reshape(BB, N, DH)

    cidx = cidx_ref[...]
    mask = (cidx > 0).astype(jnp.float32)
    n_c = jnp.sum(mask, axis=-1, keepdims=True)
    pos = lax.broadcasted_iota(jnp.int32, (1, 1, C), 2).astype(jnp.float32) + 1.0
    denom = jnp.maximum(n_c - 1.0, 1.0)
    ratio = (pos - 1.0) / denom
    ratio = jnp.where(n_c == 1.0, 0.5, ratio)
    eta_r = (mask * ratio).astype(jnp.bfloat16)
    eta_l = (mask * (1.0 - ratio)).astype(jnp.bfloat16)

    cidx_bf = cidx.astype(jnp.bfloat16)
    iota_n = lax.broadcasted_iota(jnp.int32, (N, N), 1).astype(jnp.bfloat16)
    zero = jnp.zeros((N, N), jnp.bfloat16)
    for bb in range(BB):
        a_l = zero
        a_r = zero
        for c in range(C):
            hit = cidx_bf[bb][:, c:c + 1] == iota_n
            a_l = a_l + jnp.where(hit, eta_l[bb][:, c:c + 1], zero)
            a_r = a_r + jnp.where(hit, eta_r[bb][:, c:c + 1], zero)
        alr_ref[bb, 0:N, :] = a_l
        alr_ref[bb, N:2 * N, :] = a_r


def _tc_main_body(tok_ref, te_ref, alr_ref, lab_ref, wcat_ref, cb_ref,
                  sw_ref, sb_ref, logits_ref, loss_ref):
    i = pl.program_id(0)
    ne3 = jnp.concatenate([te_ref[...], tok_ref[...].astype(jnp.bfloat16)],
                          axis=-1)

    parts = []
    for bb in range(BB):
        agg = jnp.dot(alr_ref[bb], ne3[bb], preferred_element_type=jnp.float32)
        parts.append(jnp.concatenate(
            [ne3[bb], agg[:N].astype(jnp.bfloat16),
             agg[N:].astype(jnp.bfloat16)], axis=1))
    x_all = jnp.concatenate(parts, axis=0)

    h = (jnp.dot(x_all, wcat_ref[...], preferred_element_type=jnp.float32)
         + cb_ref[...])
    conv = jnp.tanh(h)
    cv = jnp.max(conv.reshape(BB, N, D), axis=1)

    logits = (lax.dot_general(cv, sw_ref[...], (((1,), (1,)), ((), ())),
                              preferred_element_type=jnp.float32)
              + sb_ref[...])
    logits_ref[...] = logits

    lab = lab_ref[...]
    bce = (jnp.maximum(logits, 0.0) - logits * lab
           + jnp.log1p(jnp.exp(-jnp.abs(logits))))
    partial = jnp.sum(jnp.sum(bce, axis=1, keepdims=True), axis=0, keepdims=True)

    @pl.when(i == 0)
    def _():
        loss_ref[...] = jnp.zeros((1, 1), jnp.float32)

    loss_ref[...] += partial

    @pl.when(i == GRID - 1)
    def _():
        loss_ref[...] = loss_ref[...] * (1.0 / (B * NUM_SUB))


def _tc_pre_call(node_type, children_index, ttab_pad):
    return pl.pallas_call(
        _tc_pre_body,
        grid=(GRID,),
        in_specs=[
            pl.BlockSpec((BB, N), lambda i: (i, 0)),
            pl.BlockSpec((BB, N, C), lambda i: (i, 0, 0)),
            pl.BlockSpec((DH, DH), lambda i: (0, 0)),
        ],
        out_specs=[
            pl.BlockSpec((BB, N, DH), lambda i: (i, 0, 0)),
            pl.BlockSpec((BB, 2 * N, N), lambda i: (i, 0, 0)),
        ],
        out_shape=[
            jax.ShapeDtypeStruct((B, N, DH), jnp.bfloat16),
            jax.ShapeDtypeStruct((B, 2 * N, N), jnp.bfloat16),
        ],
    )(node_type, children_index, ttab_pad)


def _tc_main_call(tok_emb, te, alr, subtree_labels, w_cat, conv_b,
                  softmax_w, softmax_b):
    return pl.pallas_call(
        _tc_main_body,
        grid=(GRID,),
        in_specs=[
            pl.BlockSpec((BB, N, DH), lambda i: (i, 0, 0)),
            pl.BlockSpec((BB, N, DH), lambda i: (i, 0, 0)),
            pl.BlockSpec((BB, 2 * N, N), lambda i: (i, 0, 0)),
            pl.BlockSpec((BB, NUM_SUB), lambda i: (i, 0)),
            pl.BlockSpec((3 * D, D), lambda i: (0, 0)),
            pl.BlockSpec((1, D), lambda i: (0, 0)),
            pl.BlockSpec((NUM_SUB, D), lambda i: (0, 0)),
            pl.BlockSpec((1, NUM_SUB), lambda i: (0, 0)),
        ],
        out_specs=[
            pl.BlockSpec((BB, NUM_SUB), lambda i: (i, 0)),
            pl.BlockSpec((1, 1), lambda i: (0, 0)),
        ],
        out_shape=[
            jax.ShapeDtypeStruct((B, NUM_SUB), jnp.float32),
            jax.ShapeDtypeStruct((1, 1), jnp.float32),
        ],
    )(tok_emb, te, alr, subtree_labels, w_cat, conv_b, softmax_w, softmax_b)


def kernel(node_type, node_tokens, children_index, subtree_labels,
           type_table, token_table, Wt, Wl, Wr, conv_b, softmax_w, softmax_b):
    node_type = node_type.astype(jnp.int32)
    children_index = children_index.astype(jnp.int32)
    tok2d = node_tokens.astype(jnp.int32).reshape(M * T // 128, 128)

    tok_emb = _sc_token_mean(token_table, tok2d)
    tok_emb = tok_emb.reshape(B, N, DH)

    ttab_pad = jnp.zeros((DH, DH), jnp.float32).at[:type_table.shape[0]].set(
        type_table).astype(jnp.bfloat16)
    w_cat = jnp.concatenate([Wt, Wl, Wr], axis=0).astype(jnp.bfloat16)

    te, alr = _tc_pre_call(node_type, children_index, ttab_pad)
    logits, loss = _tc_main_call(
        tok_emb, te, alr, subtree_labels, w_cat, conv_b.reshape(1, D),
        softmax_w, softmax_b.reshape(1, NUM_SUB))
    return (loss[0, 0], logits)


# speedup vs baseline: 25.3315x; 1.2660x over previous
import functools

import jax
import jax.numpy as jnp
import numpy as np
from jax import lax
from jax.experimental import pallas as pl
from jax.experimental.pallas import tpu as pltpu
from jax.experimental.pallas import tpu_sc as plsc

B, N, C, T = 64, 256, 8, 8
DH = 128
D = 256
NUM_SUB = 5000
M = B * N

NC, NS = 2, 16
NW = NC * NS
NODES_PER_W = M // NW
NODES_PER_STEP = 128 // T
STEPS = NODES_PER_W // NODES_PER_STEP


def _sc_token_mean(token_table, tok2d):
    mesh = plsc.VectorSubcoreMesh(core_axis_name="c", subcore_axis_name="s")

    @functools.partial(
        pl.kernel,
        out_type=jax.ShapeDtypeStruct((M, DH), jnp.float32),
        mesh=mesh,
        scratch_types=[
            pltpu.VMEM((STEPS, 128), jnp.int32),
            pltpu.VMEM((2, 128, DH), jnp.float32),
            pltpu.VMEM((2, NODES_PER_STEP, DH), jnp.float32),
            pltpu.SemaphoreType.DMA,
            pltpu.SemaphoreType.DMA,
        ],
    )
    def k(table_hbm, tok_hbm, out_hbm, idx_all, rows_v, out_v, sem0, sem1):
        wid = lax.axis_index("s") * NC + lax.axis_index("c")
        base_node = wid * NODES_PER_W
        sems = (sem0, sem1)

        pltpu.sync_copy(tok_hbm.at[pl.ds(wid * STEPS, STEPS)], idx_all)

        def start_gather(s, buf):
            pltpu.async_copy(table_hbm.at[idx_all.at[s]],
                             rows_v.at[buf], sems[buf])

        def wait_gather(s, buf):
            pltpu.make_async_copy(table_hbm.at[idx_all.at[s]],
                                  rows_v.at[buf], sems[buf]).wait()

        def reduce_and_store(s, buf):
            def node(g, carry):
                for j in range(DH // 16):
                    sl = pl.ds(j * 16, 16)
                    acc = rows_v[buf, g * T, sl]
                    for t in range(1, T):
                        acc = acc + rows_v[buf, g * T + t, sl]
                    out_v[buf, g, sl] = acc * (1.0 / T)
                return carry
            lax.fori_loop(0, NODES_PER_STEP, node, 0)
            pltpu.sync_copy(
                out_v.at[buf],
                out_hbm.at[pl.ds(base_node + s * NODES_PER_STEP,
                                 NODES_PER_STEP)])

        start_gather(0, 0)

        def pair(p, carry):
            s0 = p * 2
            start_gather(s0 + 1, 1)
            wait_gather(s0, 0)
            reduce_and_store(s0, 0)

            @pl.when(s0 + 2 < STEPS)
            def _():
                start_gather(s0 + 2, 0)

            wait_gather(s0 + 1, 1)
            reduce_and_store(s0 + 1, 1)
            return carry

        lax.fori_loop(0, STEPS // 2, pair, 0)

    return k(token_table, tok2d)


BB = 8
GRID = B // BB


def _tc_pre_body(ntype_ref, cidx_ref, ttab_ref, te_ref, alr_ref):
    ntype = ntype_ref[...][..., None]
    oh = (ntype == lax.broadcasted_iota(jnp.int32, (1, 1, DH), 2)
          ).astype(jnp.bfloat16).reshape(BB * N, DH)
    te = jnp.dot(oh, ttab_ref[...], preferred_element_type=jnp.float32)
    te_ref[...] = te.astype(jnp.bfloat16).reshape(BB, N, DH)

    cidx = cidx_ref[...]
    mask = (cidx > 0).astype(jnp.float32)
    n_c = jnp.sum(mask, axis=-1, keepdims=True)
    pos = lax.broadcasted_iota(jnp.int32, (1, 1, C), 2).astype(jnp.float32) + 1.0
    denom = jnp.maximum(n_c - 1.0, 1.0)
    ratio = (pos - 1.0) / denom
    ratio = jnp.where(n_c == 1.0, 0.5, ratio)
    eta_r = (mask * ratio).astype(jnp.bfloat16)
    eta_l = (mask * (1.0 - ratio)).astype(jnp.bfloat16)

    cidx_bf = cidx.astype(jnp.bfloat16)
    iota_n = lax.broadcasted_iota(jnp.int32, (N, N), 1).astype(jnp.bfloat16)
    zero = jnp.zeros((N, N), jnp.bfloat16)
    for bb in range(BB):
        a_l = zero
        a_r = zero
        for c in range(C):
            hit = cidx_bf[bb][:, c:c + 1] == iota_n
            a_l = a_l + jnp.where(hit, eta_l[bb][:, c:c + 1], zero)
            a_r = a_r + jnp.where(hit, eta_r[bb][:, c:c + 1], zero)
        alr_ref[bb, 0:N, :] = a_l
        alr_ref[bb, N:2 * N, :] = a_r


def _tc_main_body(tok_ref, te_ref, alr_ref, lab_ref, wcat_ref, cb_ref,
                  sw_ref, sb_ref, logits_ref, loss_ref):
    i = pl.program_id(0)
    ne3 = jnp.concatenate([te_ref[...], tok_ref[...].astype(jnp.bfloat16)],
                          axis=-1)

    parts = []
    for bb in range(BB):
        agg = jnp.dot(alr_ref[bb], ne3[bb], preferred_element_type=jnp.float32)
        parts.append(jnp.concatenate(
            [ne3[bb], agg[:N].astype(jnp.bfloat16),
             agg[N:].astype(jnp.bfloat16)], axis=1))
    x_all = jnp.concatenate(parts, axis=0)

    h = (jnp.dot(x_all, wcat_ref[...], preferred_element_type=jnp.float32)
         + cb_ref[...])
    conv = jnp.tanh(h)
    cv = jnp.max(conv.reshape(BB, N, D), axis=1)

    logits = (lax.dot_general(cv, sw_ref[...], (((1,), (1,)), ((), ())),
                              preferred_element_type=jnp.float32)
              + sb_ref[...])
    logits_ref[...] = logits

    lab = lab_ref[...]
    bce = (jnp.maximum(logits, 0.0) - logits * lab
           + jnp.log1p(jnp.exp(-jnp.abs(logits))))
    partial = jnp.sum(jnp.sum(bce, axis=1, keepdims=True), axis=0, keepdims=True)

    @pl.when(i == 0)
    def _():
        loss_ref[...] = jnp.zeros((1, 1), jnp.float32)

    loss_ref[...] += partial

    @pl.when(i == GRID - 1)
    def _():
        loss_ref[...] = loss_ref[...] * (1.0 / (B * NUM_SUB))


def _tc_pre_call(node_type, children_index, ttab_pad):
    return pl.pallas_call(
        _tc_pre_body,
        grid=(GRID,),
        in_specs=[
            pl.BlockSpec((BB, N), lambda i: (i, 0)),
            pl.BlockSpec((BB, N, C), lambda i: (i, 0, 0)),
            pl.BlockSpec((DH, DH), lambda i: (0, 0)),
        ],
        out_specs=[
            pl.BlockSpec((BB, N, DH), lambda i: (i, 0, 0)),
            pl.BlockSpec((BB, 2 * N, N), lambda i: (i, 0, 0)),
        ],
        out_shape=[
            jax.ShapeDtypeStruct((B, N, DH), jnp.bfloat16),
            jax.ShapeDtypeStruct((B, 2 * N, N), jnp.bfloat16),
        ],
    )(node_type, children_index, ttab_pad)


def _tc_main_call(tok_emb, te, alr, subtree_labels, w_cat, conv_b,
                  softmax_w, softmax_b):
    return pl.pallas_call(
        _tc_main_body,
        grid=(GRID,),
        in_specs=[
            pl.BlockSpec((BB, N, DH), lambda i: (i, 0, 0)),
            pl.BlockSpec((BB, N, DH), lambda i: (i, 0, 0)),
            pl.BlockSpec((BB, 2 * N, N), lambda i: (i, 0, 0)),
            pl.BlockSpec((BB, NUM_SUB), lambda i: (i, 0)),
            pl.BlockSpec((3 * D, D), lambda i: (0, 0)),
            pl.BlockSpec((1, D), lambda i: (0, 0)),
            pl.BlockSpec((NUM_SUB, D), lambda i: (0, 0)),
            pl.BlockSpec((1, NUM_SUB), lambda i: (0, 0)),
        ],
        out_specs=[
            pl.BlockSpec((BB, NUM_SUB), lambda i: (i, 0)),
            pl.BlockSpec((1, 1), lambda i: (0, 0)),
        ],
        out_shape=[
            jax.ShapeDtypeStruct((B, NUM_SUB), jnp.float32),
            jax.ShapeDtypeStruct((1, 1), jnp.float32),
        ],
    )(tok_emb, te, alr, subtree_labels, w_cat, conv_b, softmax_w, softmax_b)


def kernel(node_type, node_tokens, children_index, subtree_labels,
           type_table, token_table, Wt, Wl, Wr, conv_b, softmax_w, softmax_b):
    node_type = node_type.astype(jnp.int32)
    children_index = children_index.astype(jnp.int32)
    tok2d = node_tokens.astype(jnp.int32).reshape(M * T // 128, 128)

    tok_emb = _sc_token_mean(token_table, tok2d)
    tok_emb = tok_emb.reshape(B, N, DH)

    ttab_pad = jnp.zeros((DH, DH), jnp.float32).at[:type_table.shape[0]].set(
        type_table).astype(jnp.bfloat16)
    w_cat = jnp.concatenate([Wt, Wl, Wr], axis=0).astype(jnp.bfloat16)

    te, alr = _tc_pre_call(node_type, children_index, ttab_pad)
    logits, loss = _tc_main_call(
        tok_emb, te, alr, subtree_labels, w_cat, conv_b.reshape(1, D),
        softmax_w, softmax_b.reshape(1, NUM_SUB))
    return (loss[0, 0], logits)


# speedup vs baseline: 26.0438x; 1.0281x over previous
import functools

import jax
import jax.numpy as jnp
import numpy as np
from jax import lax
from jax.experimental import pallas as pl
from jax.experimental.pallas import tpu as pltpu
from jax.experimental.pallas import tpu_sc as plsc

B, N, C, T = 64, 256, 8, 8
DH = 128
D = 256
NUM_SUB = 5000
M = B * N

NC, NS = 2, 16
NW = NC * NS
NODES_PER_W = M // NW
NODES_PER_STEP = 128 // T
STEPS = NODES_PER_W // NODES_PER_STEP


def _sc_token_mean(token_table, tok2d):
    mesh = plsc.VectorSubcoreMesh(core_axis_name="c", subcore_axis_name="s")

    @functools.partial(
        pl.kernel,
        out_type=jax.ShapeDtypeStruct((M, DH), jnp.float32),
        mesh=mesh,
        scratch_types=[
            pltpu.VMEM((STEPS, 128), jnp.int32),
            pltpu.VMEM((2, 128, DH), jnp.float32),
            pltpu.VMEM((2, NODES_PER_STEP, DH), jnp.float32),
            pltpu.SemaphoreType.DMA,
            pltpu.SemaphoreType.DMA,
            pltpu.SemaphoreType.DMA,
            pltpu.SemaphoreType.DMA,
        ],
    )
    def k(table_hbm, tok_hbm, out_hbm, idx_all, rows_v, out_v,
          sem0, sem1, osem0, osem1):
        wid = lax.axis_index("s") * NC + lax.axis_index("c")
        base_node = wid * NODES_PER_W
        sems = (sem0, sem1)
        osems = (osem0, osem1)

        pltpu.sync_copy(tok_hbm.at[pl.ds(wid * STEPS, STEPS)], idx_all)

        def start_gather(s, buf):
            pltpu.async_copy(table_hbm.at[idx_all.at[s]],
                             rows_v.at[buf], sems[buf])

        def wait_gather(s, buf):
            pltpu.make_async_copy(table_hbm.at[idx_all.at[s]],
                                  rows_v.at[buf], sems[buf]).wait()

        def reduce_and_store(s, buf):
            def node(g, carry):
                for j in range(DH // 16):
                    sl = pl.ds(j * 16, 16)
                    acc = rows_v[buf, g * T, sl]
                    for t in range(1, T):
                        acc = acc + rows_v[buf, g * T + t, sl]
                    out_v[buf, g, sl] = acc * (1.0 / T)
                return carry
            lax.fori_loop(0, NODES_PER_STEP, node, 0)
            pltpu.async_copy(
                out_v.at[buf],
                out_hbm.at[pl.ds(base_node + s * NODES_PER_STEP,
                                 NODES_PER_STEP)], osems[buf])

        def wait_out(s, buf):
            pltpu.make_async_copy(
                out_v.at[buf],
                out_hbm.at[pl.ds(base_node + s * NODES_PER_STEP,
                                 NODES_PER_STEP)], osems[buf]).wait()

        start_gather(0, 0)

        def pair(p, carry):
            s0 = p * 2
            start_gather(s0 + 1, 1)
            wait_gather(s0, 0)

            @pl.when(p >= 1)
            def _():
                wait_out(s0 - 2, 0)

            reduce_and_store(s0, 0)

            @pl.when(s0 + 2 < STEPS)
            def _():
                start_gather(s0 + 2, 0)

            wait_gather(s0 + 1, 1)

            @pl.when(p >= 1)
            def _():
                wait_out(s0 - 1, 1)

            reduce_and_store(s0 + 1, 1)
            return carry

        lax.fori_loop(0, STEPS // 2, pair, 0)
        wait_out(STEPS - 2, 0)
        wait_out(STEPS - 1, 1)

    return k(token_table, tok2d)


BB = 8
GRID = B // BB


def _tc_pre_body(ntype_ref, cidx_ref, ttab_ref, te_ref, alr_ref):
    ntype = ntype_ref[...][..., None]
    oh = (ntype == lax.broadcasted_iota(jnp.int32, (1, 1, DH), 2)
          ).astype(jnp.bfloat16).reshape(BB * N, DH)
    te = jnp.dot(oh, ttab_ref[...], preferred_element_type=jnp.float32)
    te_ref[...] = te.astype(jnp.bfloat16).reshape(BB, N, DH)

    cidx = cidx_ref[...]
    mask = (cidx > 0).astype(jnp.float32)
    n_c = jnp.sum(mask, axis=-1, keepdims=True)
    pos = lax.broadcasted_iota(jnp.int32, (1, 1, C), 2).astype(jnp.float32) + 1.0
    denom = jnp.maximum(n_c - 1.0, 1.0)
    ratio = (pos - 1.0) / denom
    ratio = jnp.where(n_c == 1.0, 0.5, ratio)
    eta_r = (mask * ratio).astype(jnp.bfloat16)
    eta_l = (mask * (1.0 - ratio)).astype(jnp.bfloat16)

    cidx_bf = cidx.astype(jnp.bfloat16)
    iota_n = lax.broadcasted_iota(jnp.int32, (N, N), 1).astype(jnp.bfloat16)
    zero = jnp.zeros((N, N), jnp.bfloat16)
    for bb in range(BB):
        a_l = zero
        a_r = zero
        for c in range(C):
            hit = cidx_bf[bb][:, c:c + 1] == iota_n
            a_l = a_l + jnp.where(hit, eta_l[bb][:, c:c + 1], zero)
            a_r = a_r + jnp.where(hit, eta_r[bb][:, c:c + 1], zero)
        alr_ref[bb, 0:N, :] = a_l
        alr_ref[bb, N:2 * N, :] = a_r


def _tc_main_body(tok_ref, te_ref, alr_ref, lab_ref, wcat_ref, cb_ref,
                  sw_ref, sb_ref, logits_ref, loss_ref):
    i = pl.program_id(0)
    ne3 = jnp.concatenate([te_ref[...], tok_ref[...].astype(jnp.bfloat16)],
                          axis=-1)

    parts = []
    for bb in range(BB):
        agg = jnp.dot(alr_ref[bb], ne3[bb], preferred_element_type=jnp.float32)
        parts.append(jnp.concatenate(
            [ne3[bb], agg[:N].astype(jnp.bfloat16),
             agg[N:].astype(jnp.bfloat16)], axis=1))
    x_all = jnp.concatenate(parts, axis=0)

    h = (jnp.dot(x_all, wcat_ref[...], preferred_element_type=jnp.float32)
         + cb_ref[...])
    conv = jnp.tanh(h)
    cv = jnp.max(conv.reshape(BB, N, D), axis=1)

    logits = (lax.dot_general(cv, sw_ref[...], (((1,), (1,)), ((), ())),
                              preferred_element_type=jnp.float32)
              + sb_ref[...])
    logits_ref[...] = logits

    lab = lab_ref[...]
    bce = (jnp.maximum(logits, 0.0) - logits * lab
           + jnp.log1p(jnp.exp(-jnp.abs(logits))))
    partial = jnp.sum(jnp.sum(bce, axis=1, keepdims=True), axis=0, keepdims=True)

    @pl.when(i == 0)
    def _():
        loss_ref[...] = jnp.zeros((1, 1), jnp.float32)

    loss_ref[...] += partial

    @pl.when(i == GRID - 1)
    def _():
        loss_ref[...] = loss_ref[...] * (1.0 / (B * NUM_SUB))


def _tc_pre_call(node_type, children_index, ttab_pad):
    return pl.pallas_call(
        _tc_pre_body,
        grid=(GRID,),
        in_specs=[
            pl.BlockSpec((BB, N), lambda i: (i, 0)),
            pl.BlockSpec((BB, N, C), lambda i: (i, 0, 0)),
            pl.BlockSpec((DH, DH), lambda i: (0, 0)),
        ],
        out_specs=[
            pl.BlockSpec((BB, N, DH), lambda i: (i, 0, 0)),
            pl.BlockSpec((BB, 2 * N, N), lambda i: (i, 0, 0)),
        ],
        out_shape=[
            jax.ShapeDtypeStruct((B, N, DH), jnp.bfloat16),
            jax.ShapeDtypeStruct((B, 2 * N, N), jnp.bfloat16),
        ],
    )(node_type, children_index, ttab_pad)


def _tc_main_call(tok_emb, te, alr, subtree_labels, w_cat, conv_b,
                  softmax_w, softmax_b):
    return pl.pallas_call(
        _tc_main_body,
        grid=(GRID,),
        in_specs=[
            pl.BlockSpec((BB, N, DH), lambda i: (i, 0, 0)),
            pl.BlockSpec((BB, N, DH), lambda i: (i, 0, 0)),
            pl.BlockSpec((BB, 2 * N, N), lambda i: (i, 0, 0)),
            pl.BlockSpec((BB, NUM_SUB), lambda i: (i, 0)),
            pl.BlockSpec((3 * D, D), lambda i: (0, 0)),
            pl.BlockSpec((1, D), lambda i: (0, 0)),
            pl.BlockSpec((NUM_SUB, D), lambda i: (0, 0)),
            pl.BlockSpec((1, NUM_SUB), lambda i: (0, 0)),
        ],
        out_specs=[
            pl.BlockSpec((BB, NUM_SUB), lambda i: (i, 0)),
            pl.BlockSpec((1, 1), lambda i: (0, 0)),
        ],
        out_shape=[
            jax.ShapeDtypeStruct((B, NUM_SUB), jnp.float32),
            jax.ShapeDtypeStruct((1, 1), jnp.float32),
        ],
    )(tok_emb, te, alr, subtree_labels, w_cat, conv_b, softmax_w, softmax_b)


def kernel(node_type, node_tokens, children_index, subtree_labels,
           type_table, token_table, Wt, Wl, Wr, conv_b, softmax_w, softmax_b):
    node_type = node_type.astype(jnp.int32)
    children_index = children_index.astype(jnp.int32)
    tok2d = node_tokens.astype(jnp.int32).reshape(M * T // 128, 128)

    tok_emb = _sc_token_mean(token_table, tok2d)
    tok_emb = tok_emb.reshape(B, N, DH)

    ttab_pad = jnp.zeros((DH, DH), jnp.float32).at[:type_table.shape[0]].set(
        type_table).astype(jnp.bfloat16)
    w_cat = jnp.concatenate([Wt, Wl, Wr], axis=0).astype(jnp.bfloat16)

    te, alr = _tc_pre_call(node_type, children_index, ttab_pad)
    logits, loss = _tc_main_call(
        tok_emb, te, alr, subtree_labels, w_cat, conv_b.reshape(1, D),
        softmax_w, softmax_b.reshape(1, NUM_SUB))
    return (loss[0, 0], logits)


# speedup vs baseline: 27.5964x; 1.0596x over previous
import functools

import jax
import jax.numpy as jnp
import numpy as np
from jax import lax
from jax.experimental import pallas as pl
from jax.experimental.pallas import tpu as pltpu
from jax.experimental.pallas import tpu_sc as plsc

B, N, C, T = 64, 256, 8, 8
DH = 128
D = 256
NUM_SUB = 5000
M = B * N

NC, NS = 2, 16
NW = NC * NS
NODES_PER_W = M // NW
NODES_PER_STEP = 128 // T
STEPS = NODES_PER_W // NODES_PER_STEP


def _sc_token_mean(token_table, tok2d):
    mesh = plsc.VectorSubcoreMesh(core_axis_name="c", subcore_axis_name="s")

    @functools.partial(
        pl.kernel,
        out_type=jax.ShapeDtypeStruct((M, DH), jnp.float32),
        mesh=mesh,
        scratch_types=[
            pltpu.VMEM((STEPS, 128), jnp.int32),
            pltpu.VMEM((2, 128, DH), jnp.float32),
            pltpu.VMEM((2, NODES_PER_STEP, DH), jnp.float32),
            pltpu.SemaphoreType.DMA,
            pltpu.SemaphoreType.DMA,
            pltpu.SemaphoreType.DMA,
            pltpu.SemaphoreType.DMA,
        ],
    )
    def k(table_hbm, tok_hbm, out_hbm, idx_all, rows_v, out_v,
          sem0, sem1, osem0, osem1):
        wid = lax.axis_index("s") * NC + lax.axis_index("c")
        base_node = wid * NODES_PER_W
        sems = (sem0, sem1)
        osems = (osem0, osem1)

        pltpu.sync_copy(tok_hbm.at[pl.ds(wid * STEPS, STEPS)], idx_all)

        def start_gather(s, buf):
            pltpu.async_copy(table_hbm.at[idx_all.at[s]],
                             rows_v.at[buf], sems[buf])

        def wait_gather(s, buf):
            pltpu.make_async_copy(table_hbm.at[idx_all.at[s]],
                                  rows_v.at[buf], sems[buf]).wait()

        def reduce_and_store(s, buf):
            def node(g, carry):
                for j in range(DH // 16):
                    sl = pl.ds(j * 16, 16)
                    acc = rows_v[buf, g * T, sl]
                    for t in range(1, T):
                        acc = acc + rows_v[buf, g * T + t, sl]
                    out_v[buf, g, sl] = acc * (1.0 / T)
                return carry
            lax.fori_loop(0, NODES_PER_STEP, node, 0)
            pltpu.async_copy(
                out_v.at[buf],
                out_hbm.at[pl.ds(base_node + s * NODES_PER_STEP,
                                 NODES_PER_STEP)], osems[buf])

        def wait_out(s, buf):
            pltpu.make_async_copy(
                out_v.at[buf],
                out_hbm.at[pl.ds(base_node + s * NODES_PER_STEP,
                                 NODES_PER_STEP)], osems[buf]).wait()

        start_gather(0, 0)

        def pair(p, carry):
            s0 = p * 2
            start_gather(s0 + 1, 1)
            wait_gather(s0, 0)

            @pl.when(p >= 1)
            def _():
                wait_out(s0 - 2, 0)

            reduce_and_store(s0, 0)

            @pl.when(s0 + 2 < STEPS)
            def _():
                start_gather(s0 + 2, 0)

            wait_gather(s0 + 1, 1)

            @pl.when(p >= 1)
            def _():
                wait_out(s0 - 1, 1)

            reduce_and_store(s0 + 1, 1)
            return carry

        lax.fori_loop(0, STEPS // 2, pair, 0)
        wait_out(STEPS - 2, 0)
        wait_out(STEPS - 1, 1)

    return k(token_table, tok2d)


BB = 16
GRID = B // BB


def _tc_pre_body(ntype_ref, cidx_ref, ttab_ref, te_ref, alr_ref):
    ntype = ntype_ref[...][..., None]
    oh = (ntype == lax.broadcasted_iota(jnp.int32, (1, 1, DH), 2)
          ).astype(jnp.bfloat16).reshape(BB * N, DH)
    te = jnp.dot(oh, ttab_ref[...], preferred_element_type=jnp.float32)
    te_ref[...] = te.astype(jnp.bfloat16).reshape(BB, N, DH)

    cidx = cidx_ref[...]
    mask = (cidx > 0).astype(jnp.float32)
    n_c = jnp.sum(mask, axis=-1, keepdims=True)
    pos = lax.broadcasted_iota(jnp.int32, (1, 1, C), 2).astype(jnp.float32) + 1.0
    denom = jnp.maximum(n_c - 1.0, 1.0)
    ratio = (pos - 1.0) / denom
    ratio = jnp.where(n_c == 1.0, 0.5, ratio)
    eta_r = (mask * ratio).astype(jnp.bfloat16)
    eta_l = (mask * (1.0 - ratio)).astype(jnp.bfloat16)

    cidx_bf = cidx.astype(jnp.bfloat16)
    iota_n = lax.broadcasted_iota(jnp.int32, (N, N), 1).astype(jnp.bfloat16)
    zero = jnp.zeros((N, N), jnp.bfloat16)
    for bb in range(BB):
        a_l = zero
        a_r = zero
        for c in range(C):
            hit = cidx_bf[bb][:, c:c + 1] == iota_n
            a_l = a_l + jnp.where(hit, eta_l[bb][:, c:c + 1], zero)
            a_r = a_r + jnp.where(hit, eta_r[bb][:, c:c + 1], zero)
        alr_ref[bb, 0:N, :] = a_l
        alr_ref[bb, N:2 * N, :] = a_r


def _tc_main_body(tok_ref, te_ref, alr_ref, lab_ref, wcat_ref, cb_ref,
                  sw_ref, sb_ref, logits_ref, loss_ref):
    i = pl.program_id(0)
    ne3 = jnp.concatenate([te_ref[...], tok_ref[...].astype(jnp.bfloat16)],
                          axis=-1)

    parts = []
    for bb in range(BB):
        agg = jnp.dot(alr_ref[bb], ne3[bb], preferred_element_type=jnp.float32)
        parts.append(jnp.concatenate(
            [ne3[bb], agg[:N].astype(jnp.bfloat16),
             agg[N:].astype(jnp.bfloat16)], axis=1))
    x_all = jnp.concatenate(parts, axis=0)

    h = (jnp.dot(x_all, wcat_ref[...], preferred_element_type=jnp.float32)
         + cb_ref[...])
    conv = jnp.tanh(h)
    cv = jnp.max(conv.reshape(BB, N, D), axis=1)

    logits = (lax.dot_general(cv, sw_ref[...], (((1,), (1,)), ((), ())),
                              preferred_element_type=jnp.float32)
              + sb_ref[...])
    logits_ref[...] = logits

    lab = lab_ref[...]
    bce = (jnp.maximum(logits, 0.0) - logits * lab
           + jnp.log1p(jnp.exp(-jnp.abs(logits))))
    partial = jnp.sum(jnp.sum(bce, axis=1, keepdims=True), axis=0, keepdims=True)

    @pl.when(i == 0)
    def _():
        loss_ref[...] = jnp.zeros((1, 1), jnp.float32)

    loss_ref[...] += partial

    @pl.when(i == GRID - 1)
    def _():
        loss_ref[...] = loss_ref[...] * (1.0 / (B * NUM_SUB))


def _tc_pre_call(node_type, children_index, ttab_pad):
    return pl.pallas_call(
        _tc_pre_body,
        grid=(GRID,),
        in_specs=[
            pl.BlockSpec((BB, N), lambda i: (i, 0)),
            pl.BlockSpec((BB, N, C), lambda i: (i, 0, 0)),
            pl.BlockSpec((DH, DH), lambda i: (0, 0)),
        ],
        out_specs=[
            pl.BlockSpec((BB, N, DH), lambda i: (i, 0, 0)),
            pl.BlockSpec((BB, 2 * N, N), lambda i: (i, 0, 0)),
        ],
        out_shape=[
            jax.ShapeDtypeStruct((B, N, DH), jnp.bfloat16),
            jax.ShapeDtypeStruct((B, 2 * N, N), jnp.bfloat16),
        ],
    )(node_type, children_index, ttab_pad)


def _tc_main_call(tok_emb, te, alr, subtree_labels, w_cat, conv_b,
                  softmax_w, softmax_b):
    return pl.pallas_call(
        _tc_main_body,
        grid=(GRID,),
        in_specs=[
            pl.BlockSpec((BB, N, DH), lambda i: (i, 0, 0)),
            pl.BlockSpec((BB, N, DH), lambda i: (i, 0, 0)),
            pl.BlockSpec((BB, 2 * N, N), lambda i: (i, 0, 0)),
            pl.BlockSpec((BB, NUM_SUB), lambda i: (i, 0)),
            pl.BlockSpec((3 * D, D), lambda i: (0, 0)),
            pl.BlockSpec((1, D), lambda i: (0, 0)),
            pl.BlockSpec((NUM_SUB, D), lambda i: (0, 0)),
            pl.BlockSpec((1, NUM_SUB), lambda i: (0, 0)),
        ],
        out_specs=[
            pl.BlockSpec((BB, NUM_SUB), lambda i: (i, 0)),
            pl.BlockSpec((1, 1), lambda i: (0, 0)),
        ],
        out_shape=[
            jax.ShapeDtypeStruct((B, NUM_SUB), jnp.float32),
            jax.ShapeDtypeStruct((1, 1), jnp.float32),
        ],
    )(tok_emb, te, alr, subtree_labels, w_cat, conv_b, softmax_w, softmax_b)


def kernel(node_type, node_tokens, children_index, subtree_labels,
           type_table, token_table, Wt, Wl, Wr, conv_b, softmax_w, softmax_b):
    node_type = node_type.astype(jnp.int32)
    children_index = children_index.astype(jnp.int32)
    tok2d = node_tokens.astype(jnp.int32).reshape(M * T // 128, 128)

    tok_emb = _sc_token_mean(token_table, tok2d)
    tok_emb = tok_emb.reshape(B, N, DH)

    ttab_pad = jnp.zeros((DH, DH), jnp.float32).at[:type_table.shape[0]].set(
        type_table).astype(jnp.bfloat16)
    w_cat = jnp.concatenate([Wt, Wl, Wr], axis=0).astype(jnp.bfloat16)

    te, alr = _tc_pre_call(node_type, children_index, ttab_pad)
    logits, loss = _tc_main_call(
        tok_emb, te, alr, subtree_labels, w_cat, conv_b.reshape(1, D),
        softmax_w, softmax_b.reshape(1, NUM_SUB))
    return (loss[0, 0], logits)
